# Initial kernel scaffold; baseline (speedup 1.0000x reference)
import jax
import jax.numpy as jnp
from jax.experimental import pallas as pl


def kernel(pred_boxes, target_boxes):
    raise NotImplementedError("write your pallas kernel here")


# speedup vs baseline: 99.1771x; 99.1771x over previous
import jax
import jax.numpy as jnp
from jax.experimental import pallas as pl
from jax.experimental.pallas import tpu as pltpu

_N = 131072
_LANE = 128
_ROWS = _N // _LANE
_RB = 64
_G = _ROWS // _RB
_K = 8


def _corner_lists(x, y, w, l, im, re):
    inv = jax.lax.rsqrt(im * im + re * re)
    c = re * inv
    s = im * inv
    hw = w * 0.5
    hl = l * 0.5
    hwc = hw * c
    hws = hw * s
    hlc = hl * c
    hls = hl * s
    cx = [x - hwc - hls, x - hwc + hls, x + hwc + hls, x + hwc - hls]
    cy = [y - hws + hlc, y - hws - hlc, y + hws - hlc, y + hws + hlc]
    return cx, cy


def _clip_inter_area(sx, sy, cx, cy):
    z = jnp.zeros_like(sx[0])
    one = jnp.ones_like(z)
    px = list(sx) + [z, z, z, z]
    py = list(sy) + [z, z, z, z]
    mval = [one, one, one, one, z, z, z, z]
    m = one * 4.0
    frozen = z
    for e in range(4):
        pX, pY = cx[e], cy[e]
        qX, qY = cx[(e + 1) % 4], cy[(e + 1) % 4]
        a = qY - pY
        b = pX - qX
        c = qX * pY - qY * pX
        cap_prev = 4 + e
        cap_new = min(5 + e, _K)
        vals = [a * px[k] + b * py[k] + c for k in range(cap_prev)]
        candx, candy, kv = [], [], []
        for k in range(cap_prev):
            if e == 0:
                k2 = (k + 1) % 4
                tpx, tpy, tv = px[k2], py[k2], vals[k2]
                mk = one
            else:
                if k + 1 < cap_prev:
                    wrap = m <= (k + 1)
                    tpx = jnp.where(wrap, px[0], px[k + 1])
                    tpy = jnp.where(wrap, py[0], py[k + 1])
                    tv = jnp.where(wrap, vals[0], vals[k + 1])
                else:
                    tpx, tpy, tv = px[0], py[0], vals[0]
                mk = mval[k]
            keep01 = jnp.where(vals[k] <= 0, mk, z)
            cross01 = jnp.where(vals[k] * tv < 0, mk, z)
            cb = cross01 > 0.5
            a2 = tpy - py[k]
            b2 = px[k] - tpx
            c2 = tpx * py[k] - tpy * px[k]
            wdet = a * b2 - b * a2
            wsafe = jnp.where(cb, wdet, 1.0)
            ix = (b * c2 - c * b2) / wsafe
            iy = (c * a2 - a * c2) / wsafe
            candx += [px[k], ix]
            candy += [py[k], iy]
            kv += [keep01, cross01]
        run = z
        keys = []
        for v in kv:
            keys.append((run + 1.0) * v)
            run = run + v
        new_px, new_py = [], []
        ncand = len(kv)
        for j in range(cap_new):
            nx, ny = z, z
            for ci in range(j, ncand):
                hit = keys[ci] == (j + 1.0)
                nx = jnp.where(hit, candx[ci], nx)
                ny = jnp.where(hit, candy[ci], ny)
            new_px.append(nx)
            new_py.append(ny)
        if e == 0:
            do01 = one
        else:
            do01 = jnp.where(frozen > 0.5, z, jnp.where(m > 2.5, one, z))
        accept01 = jnp.where(run > 0.5, do01, z)
        acc_b = accept01 > 0.5
        for j in range(cap_new):
            px[j] = jnp.where(acc_b, new_px[j], px[j])
            py[j] = jnp.where(acc_b, new_py[j], py[j])
            nmv = jnp.where(run > (j + 0.5), one, z)
            mval[j] = jnp.where(acc_b, nmv, mval[j])
        m = jnp.where(acc_b, jnp.minimum(run, float(cap_new)), m)
        frozen = jnp.maximum(frozen, jnp.where(run < 0.5, do01, z))
    fx, fy = px[0], py[0]
    spx = [jnp.where(mval[k] > 0.5, px[k], fx) for k in range(_K)]
    spy = [jnp.where(mval[k] > 0.5, py[k], fy) for k in range(_K)]
    acc = z
    for k in range(_K):
        k2 = (k + 1) % _K
        acc = acc + (spx[k] * spy[k2] - spy[k] * spx[k2])
    area = 0.5 * jnp.abs(acc)
    return jnp.where(m > 2.5, area, z)


def _hull_area8(hx, hy):
    z = jnp.zeros_like(hx[0])
    cr = {}
    for a in range(_K):
        for b in range(a + 1, _K):
            cr[(a, b)] = hx[a] * hy[b] - hy[a] * hx[b]

    def _crs(a, b):
        return cr[(a, b)] if a < b else -cr[(b, a)]

    total = z
    for i in range(_K):
        for j in range(i + 1, _K):
            same = (i < 4) == (j < 4)
            if same and (j - i) % 4 == 2:
                continue
            crij = cr[(i, j)]
            if same:
                quad = range(0, 4) if i < 4 else range(4, 8)
                cycle_fwd = (j - i == 1)
                ks = [k for k in range(_K) if k not in quad]
            else:
                cycle_fwd = None
                ks = [k for k in range(_K) if k != i and k != j]
            mn = None
            mx = None
            for k in ks:
                c = crij + _crs(j, k) + _crs(k, i)
                if cycle_fwd is not False:
                    mn = c if mn is None else jnp.minimum(mn, c)
                if cycle_fwd is not True:
                    mx = c if mx is None else jnp.maximum(mx, c)
            if same:
                if cycle_fwd:
                    total = total + jnp.where(mn >= -1e-6, crij, 0.0)
                else:
                    total = total - jnp.where(-mx >= -1e-6, crij, 0.0)
            else:
                ex = hx[j] - hx[i]
                ey = hy[j] - hy[i]
                nrm_ok = (ex * ex + ey * ey) > 1e-12
                fwd = jnp.where(mn >= -1e-6, jnp.where(nrm_ok, crij, 0.0), 0.0)
                bwd = jnp.where(-mx >= -1e-6, jnp.where(nrm_ok, crij, 0.0), 0.0)
                total = total + (fwd - bwd)
    return 0.5 * jnp.abs(total)


def _giou_block(p_ref, t_ref, iou_ref, part_ref):
    xp, yp, wp, lp, imp, rep = (p_ref[i] for i in range(6))
    xt, yt, wt, lt, imt, ret = (t_ref[i] for i in range(6))
    pcx, pcy = _corner_lists(xp, yp, wp, lp, imp, rep)
    tcx, tcy = _corner_lists(xt, yt, wt, lt, imt, ret)
    inter = _clip_inter_area(pcx, pcy, tcx, tcy)
    p_area = wp * lp
    t_area = wt * lt
    union = p_area + t_area - inter
    iou = inter / (union + 1e-16)
    hull = _hull_area8(pcx + tcx, pcy + tcy)
    giou = 1.0 - (iou - (hull - union) / (hull + 1e-16))
    iou_ref[...] = iou
    part_ref[0] = jnp.sum(giou, axis=0, keepdims=True)


def kernel(pred_boxes, target_boxes):
    p = pred_boxes.T.reshape(6, _ROWS, _LANE)
    t = target_boxes.T.reshape(6, _ROWS, _LANE)
    iou2d, parts = pl.pallas_call(
        _giou_block,
        out_shape=(
            jax.ShapeDtypeStruct((_ROWS, _LANE), jnp.float32),
            jax.ShapeDtypeStruct((_G, 1, _LANE), jnp.float32),
        ),
        grid=(_G,),
        in_specs=[
            pl.BlockSpec((6, _RB, _LANE), lambda g: (0, g, 0)),
            pl.BlockSpec((6, _RB, _LANE), lambda g: (0, g, 0)),
        ],
        out_specs=(
            pl.BlockSpec((_RB, _LANE), lambda g: (g, 0)),
            pl.BlockSpec((1, 1, _LANE), lambda g: (g, 0, 0)),
        ),
        compiler_params=pltpu.CompilerParams(
            dimension_semantics=("arbitrary",),
        ),
    )(p, t)
    iou = iou2d.reshape(_N)
    giou_loss = jnp.sum(parts).reshape(1)
    return iou, giou_loss


# speedup vs baseline: 110.9015x; 1.1182x over previous
import jax
import jax.numpy as jnp
from jax.experimental import pallas as pl
from jax.experimental.pallas import tpu as pltpu

_N = 131072
_LANE = 128
_ROWS = _N // _LANE
_RB = 64
_G = _ROWS // _RB
_K = 8


def _corner_lists(x, y, w, l, im, re):
    inv = jax.lax.rsqrt(im * im + re * re)
    c = re * inv
    s = im * inv
    hw = w * 0.5
    hl = l * 0.5
    hwc = hw * c
    hws = hw * s
    hlc = hl * c
    hls = hl * s
    cx = [x - hwc - hls, x - hwc + hls, x + hwc + hls, x + hwc - hls]
    cy = [y - hws + hlc, y - hws - hlc, y + hws - hlc, y + hws + hlc]
    return cx, cy


def _clip_inter_area(sx, sy, cx, cy):
    z = jnp.zeros_like(sx[0])
    one = jnp.ones_like(z)
    px = list(sx) + [z, z, z, z]
    py = list(sy) + [z, z, z, z]
    mval = [one, one, one, one, z, z, z, z]
    m = one * 4.0
    frozen = z
    for e in range(4):
        pX, pY = cx[e], cy[e]
        qX, qY = cx[(e + 1) % 4], cy[(e + 1) % 4]
        a = qY - pY
        b = pX - qX
        c = qX * pY - qY * pX
        cap_prev = 4 + e
        cap_new = min(5 + e, _K)
        vals = [a * px[k] + b * py[k] + c for k in range(cap_prev)]
        candx, candy, kv = [], [], []
        for k in range(cap_prev):
            if e == 0:
                k2 = (k + 1) % 4
                tpx, tpy, tv = px[k2], py[k2], vals[k2]
                mk = one
            else:
                if k + 1 < cap_prev:
                    wrap = m <= (k + 1)
                    tpx = jnp.where(wrap, px[0], px[k + 1])
                    tpy = jnp.where(wrap, py[0], py[k + 1])
                    tv = jnp.where(wrap, vals[0], vals[k + 1])
                else:
                    tpx, tpy, tv = px[0], py[0], vals[0]
                mk = mval[k]
            keep01 = jnp.where(vals[k] <= 0, mk, z)
            cross01 = jnp.where(vals[k] * tv < 0, mk, z)
            cb = cross01 > 0.5
            den = jnp.where(cb, vals[k] - tv, 1.0)
            tpar = vals[k] / den
            ix = px[k] + tpar * (tpx - px[k])
            iy = py[k] + tpar * (tpy - py[k])
            candx += [px[k], ix]
            candy += [py[k], iy]
            kv += [keep01, cross01]
        ncand = len(kv)
        run = z
        for v in kv:
            run = run + v
        if e == 0:
            do01 = one
        else:
            do01 = jnp.where(frozen > 0.5, z, jnp.where(m > 2.5, one, z))
        accept01 = jnp.where(run > 0.5, do01, z)
        acc_b = accept01 > 0.5
        if e < 3:
            pos = z
            keys = []
            for v in kv:
                keys.append((pos + 1.0) * v)
                pos = pos + v
            new_px, new_py = [], []
            for j in range(cap_new):
                nx, ny = z, z
                for ci in range(j, ncand):
                    hit = keys[ci] == (j + 1.0)
                    nx = jnp.where(hit, candx[ci], nx)
                    ny = jnp.where(hit, candy[ci], ny)
                new_px.append(nx)
                new_py.append(ny)
            for j in range(cap_new):
                px[j] = jnp.where(acc_b, new_px[j], px[j])
                py[j] = jnp.where(acc_b, new_py[j], py[j])
                nmv = jnp.where(run > (j + 0.5), one, z)
                mval[j] = jnp.where(acc_b, nmv, mval[j])
            m = jnp.where(acc_b, jnp.minimum(run, float(cap_new)), m)
            frozen = jnp.maximum(frozen, jnp.where(run < 0.5, do01, z))
        else:
            kvb = [v > 0.5 for v in kv]
            fcx, fcy = z, z
            for ci in reversed(range(ncand)):
                fcx = jnp.where(kvb[ci], candx[ci], fcx)
                fcy = jnp.where(kvb[ci], candy[ci], fcy)
            carry_x, carry_y = fcx, fcy
            sacc = z
            for ci in reversed(range(ncand)):
                contrib = candx[ci] * carry_y - candy[ci] * carry_x
                sacc = sacc + jnp.where(kvb[ci], contrib, z)
                carry_x = jnp.where(kvb[ci], candx[ci], carry_x)
                carry_y = jnp.where(kvb[ci], candy[ci], carry_y)
            area_new = 0.5 * jnp.abs(sacc)
            fx, fy = px[0], py[0]
            spx = [jnp.where(mval[k] > 0.5, px[k], fx) for k in range(cap_prev)]
            spy = [jnp.where(mval[k] > 0.5, py[k], fy) for k in range(cap_prev)]
            pacc = z
            for k in range(cap_prev):
                k2 = (k + 1) % cap_prev
                pacc = pacc + (spx[k] * spy[k2] - spy[k] * spx[k2])
            area_prev = 0.5 * jnp.abs(pacc)
            m_fin = jnp.where(acc_b, jnp.minimum(run, float(cap_new)), m)
            area = jnp.where(acc_b, area_new, area_prev)
            return jnp.where(m_fin > 2.5, area, z)


def _hull_area8(hx, hy):
    z = jnp.zeros_like(hx[0])
    cr = {}
    for a in range(_K):
        for b in range(a + 1, _K):
            cr[(a, b)] = hx[a] * hy[b] - hy[a] * hx[b]

    def _crs(a, b):
        return cr[(a, b)] if a < b else -cr[(b, a)]

    total = z
    for i in range(_K):
        for j in range(i + 1, _K):
            same = (i < 4) == (j < 4)
            if same and (j - i) % 4 == 2:
                continue
            crij = cr[(i, j)]
            if same:
                quad = range(0, 4) if i < 4 else range(4, 8)
                cycle_fwd = (j - i == 1)
                ks = [k for k in range(_K) if k not in quad]
            else:
                cycle_fwd = None
                ks = [k for k in range(_K) if k != i and k != j]
            mn = None
            mx = None
            for k in ks:
                c = crij + _crs(j, k) + _crs(k, i)
                if cycle_fwd is not False:
                    mn = c if mn is None else jnp.minimum(mn, c)
                if cycle_fwd is not True:
                    mx = c if mx is None else jnp.maximum(mx, c)
            if same:
                if cycle_fwd:
                    total = total + jnp.where(mn >= -1e-6, crij, 0.0)
                else:
                    total = total - jnp.where(-mx >= -1e-6, crij, 0.0)
            else:
                ex = hx[j] - hx[i]
                ey = hy[j] - hy[i]
                nrm_ok = (ex * ex + ey * ey) > 1e-12
                fwd = jnp.where(mn >= -1e-6, jnp.where(nrm_ok, crij, 0.0), 0.0)
                bwd = jnp.where(-mx >= -1e-6, jnp.where(nrm_ok, crij, 0.0), 0.0)
                total = total + (fwd - bwd)
    return 0.5 * jnp.abs(total)


def _giou_block(p_ref, t_ref, iou_ref, part_ref):
    xp, yp, wp, lp, imp, rep = (p_ref[i] for i in range(6))
    xt, yt, wt, lt, imt, ret = (t_ref[i] for i in range(6))
    pcx, pcy = _corner_lists(xp, yp, wp, lp, imp, rep)
    tcx, tcy = _corner_lists(xt, yt, wt, lt, imt, ret)
    inter = _clip_inter_area(pcx, pcy, tcx, tcy)
    p_area = wp * lp
    t_area = wt * lt
    union = p_area + t_area - inter
    iou = inter / (union + 1e-16)
    hull = _hull_area8(pcx + tcx, pcy + tcy)
    giou = 1.0 - (iou - (hull - union) / (hull + 1e-16))
    iou_ref[...] = iou
    part_ref[0] = jnp.sum(giou, axis=0, keepdims=True)


def kernel(pred_boxes, target_boxes):
    p = pred_boxes.T.reshape(6, _ROWS, _LANE)
    t = target_boxes.T.reshape(6, _ROWS, _LANE)
    iou2d, parts = pl.pallas_call(
        _giou_block,
        out_shape=(
            jax.ShapeDtypeStruct((_ROWS, _LANE), jnp.float32),
            jax.ShapeDtypeStruct((_G, 1, _LANE), jnp.float32),
        ),
        grid=(_G,),
        in_specs=[
            pl.BlockSpec((6, _RB, _LANE), lambda g: (0, g, 0)),
            pl.BlockSpec((6, _RB, _LANE), lambda g: (0, g, 0)),
        ],
        out_specs=(
            pl.BlockSpec((_RB, _LANE), lambda g: (g, 0)),
            pl.BlockSpec((1, 1, _LANE), lambda g: (g, 0, 0)),
        ),
        compiler_params=pltpu.CompilerParams(
            dimension_semantics=("arbitrary",),
        ),
    )(p, t)
    iou = iou2d.reshape(_N)
    giou_loss = jnp.sum(parts).reshape(1)
    return iou, giou_loss


# speedup vs baseline: 115.4329x; 1.0409x over previous
import jax
import jax.numpy as jnp
from jax.experimental import pallas as pl
from jax.experimental.pallas import tpu as pltpu

_N = 131072
_LANE = 128
_ROWS = _N // _LANE
_RB = 32
_G = _ROWS // _RB
_K = 8


def _corner_lists(x, y, w, l, im, re):
    inv = jax.lax.rsqrt(im * im + re * re)
    c = re * inv
    s = im * inv
    hw = w * 0.5
    hl = l * 0.5
    hwc = hw * c
    hws = hw * s
    hlc = hl * c
    hls = hl * s
    cx = [x - hwc - hls, x - hwc + hls, x + hwc + hls, x + hwc - hls]
    cy = [y - hws + hlc, y - hws - hlc, y + hws - hlc, y + hws + hlc]
    return cx, cy


def _clip_inter_area(sx, sy, cx, cy):
    z = jnp.zeros_like(sx[0])
    one = jnp.ones_like(z)
    px = list(sx) + [z, z, z, z]
    py = list(sy) + [z, z, z, z]
    mval = [one, one, one, one, z, z, z, z]
    m = one * 4.0
    frozen = z
    for e in range(4):
        pX, pY = cx[e], cy[e]
        qX, qY = cx[(e + 1) % 4], cy[(e + 1) % 4]
        a = qY - pY
        b = pX - qX
        c = qX * pY - qY * pX
        cap_prev = 4 + e
        cap_new = min(5 + e, _K)
        vals = [a * px[k] + b * py[k] + c for k in range(cap_prev)]
        candx, candy, kv = [], [], []
        for k in range(cap_prev):
            if e == 0:
                k2 = (k + 1) % 4
                tpx, tpy, tv = px[k2], py[k2], vals[k2]
                mk = one
            else:
                if k + 1 < cap_prev:
                    wrap = m <= (k + 1)
                    tpx = jnp.where(wrap, px[0], px[k + 1])
                    tpy = jnp.where(wrap, py[0], py[k + 1])
                    tv = jnp.where(wrap, vals[0], vals[k + 1])
                else:
                    tpx, tpy, tv = px[0], py[0], vals[0]
                mk = mval[k]
            cross_cmp = vals[k] * tv < 0
            keep01 = jnp.where(vals[k] <= 0, mk, z)
            cross01 = jnp.where(cross_cmp, mk, z)
            den = jnp.where(cross_cmp, vals[k] - tv, 1.0)
            tpar = vals[k] / den
            ix = px[k] + tpar * (tpx - px[k])
            iy = py[k] + tpar * (tpy - py[k])
            candx += [px[k], ix]
            candy += [py[k], iy]
            kv += [keep01, cross01]
        ncand = len(kv)
        run = z
        for v in kv:
            run = run + v
        if e == 0:
            do01 = one
        else:
            do01 = jnp.where(frozen > 0.5, z, jnp.where(m > 2.5, one, z))
        accept01 = jnp.where(run > 0.5, do01, z)
        acc_b = accept01 > 0.5
        if e < 3:
            run_incl = z
            keys = []
            for v in kv:
                run_incl = run_incl + v
                keys.append(run_incl * v)
            new_px, new_py = [], []
            for j in range(cap_new):
                nx, ny = z, z
                for ci in range(j, ncand):
                    hit = keys[ci] == (j + 1.0)
                    nx = jnp.where(hit, candx[ci], nx)
                    ny = jnp.where(hit, candy[ci], ny)
                new_px.append(nx)
                new_py.append(ny)
            for j in range(cap_new):
                px[j] = jnp.where(acc_b, new_px[j], px[j])
                py[j] = jnp.where(acc_b, new_py[j], py[j])
                nmv = jnp.where(run > (j + 0.5), one, z)
                mval[j] = jnp.where(acc_b, nmv, mval[j])
            m = jnp.where(acc_b, jnp.minimum(run, float(cap_new)), m)
            frozen = jnp.maximum(frozen, jnp.where(run < 0.5, do01, z))
        else:
            kvb = [v > 0.5 for v in kv]
            fcx, fcy = z, z
            for ci in reversed(range(ncand)):
                fcx = jnp.where(kvb[ci], candx[ci], fcx)
                fcy = jnp.where(kvb[ci], candy[ci], fcy)
            carry_x, carry_y = fcx, fcy
            sacc = z
            for ci in reversed(range(ncand)):
                contrib = candx[ci] * carry_y - candy[ci] * carry_x
                sacc = sacc + jnp.where(kvb[ci], contrib, z)
                carry_x = jnp.where(kvb[ci], candx[ci], carry_x)
                carry_y = jnp.where(kvb[ci], candy[ci], carry_y)
            area_new = 0.5 * jnp.abs(sacc)
            fx, fy = px[0], py[0]
            spx = [jnp.where(mval[k] > 0.5, px[k], fx) for k in range(cap_prev)]
            spy = [jnp.where(mval[k] > 0.5, py[k], fy) for k in range(cap_prev)]
            pacc = z
            for k in range(cap_prev):
                k2 = (k + 1) % cap_prev
                pacc = pacc + (spx[k] * spy[k2] - spy[k] * spx[k2])
            area_prev = 0.5 * jnp.abs(pacc)
            m_fin = jnp.where(acc_b, jnp.minimum(run, float(cap_new)), m)
            area = jnp.where(acc_b, area_new, area_prev)
            return jnp.where(m_fin > 2.5, area, z)


def _hull_area8(hx, hy):
    z = jnp.zeros_like(hx[0])
    cr = {}
    for a in range(_K):
        for b in range(a + 1, _K):
            cr[(a, b)] = hx[a] * hy[b] - hy[a] * hx[b]

    def _crs(a, b):
        return cr[(a, b)] if a < b else -cr[(b, a)]

    total = z
    for i in range(_K):
        for j in range(i + 1, _K):
            same = (i < 4) == (j < 4)
            if same and (j - i) % 4 == 2:
                continue
            crij = cr[(i, j)]
            if same:
                quad = range(0, 4) if i < 4 else range(4, 8)
                cycle_fwd = (j - i == 1)
                ks = [k for k in range(_K) if k not in quad]
            else:
                cycle_fwd = None
                ks = [k for k in range(_K) if k != i and k != j]
            mn = None
            mx = None
            for k in ks:
                c = crij + _crs(j, k) + _crs(k, i)
                if cycle_fwd is not False:
                    mn = c if mn is None else jnp.minimum(mn, c)
                if cycle_fwd is not True:
                    mx = c if mx is None else jnp.maximum(mx, c)
            if same:
                if cycle_fwd:
                    total = total + jnp.where(mn >= -1e-6, crij, 0.0)
                else:
                    total = total - jnp.where(-mx >= -1e-6, crij, 0.0)
            else:
                fwd = jnp.where(mn >= -1e-6, crij, 0.0)
                bwd = jnp.where(-mx >= -1e-6, crij, 0.0)
                total = total + (fwd - bwd)
    return 0.5 * jnp.abs(total)


def _giou_block(p_ref, t_ref, iou_ref, part_ref):
    xp, yp, wp, lp, imp, rep = (p_ref[i] for i in range(6))
    xt, yt, wt, lt, imt, ret = (t_ref[i] for i in range(6))
    pcx, pcy = _corner_lists(xp, yp, wp, lp, imp, rep)
    tcx, tcy = _corner_lists(xt, yt, wt, lt, imt, ret)
    inter = _clip_inter_area(pcx, pcy, tcx, tcy)
    p_area = wp * lp
    t_area = wt * lt
    union = p_area + t_area - inter
    iou = inter / (union + 1e-16)
    hull = _hull_area8(pcx + tcx, pcy + tcy)
    giou = 1.0 - (iou - (hull - union) / (hull + 1e-16))
    iou_ref[...] = iou
    part_ref[0] = jnp.sum(giou, axis=0, keepdims=True)


def kernel(pred_boxes, target_boxes):
    p = pred_boxes.T.reshape(6, _ROWS, _LANE)
    t = target_boxes.T.reshape(6, _ROWS, _LANE)
    iou2d, parts = pl.pallas_call(
        _giou_block,
        out_shape=(
            jax.ShapeDtypeStruct((_ROWS, _LANE), jnp.float32),
            jax.ShapeDtypeStruct((_G, 1, _LANE), jnp.float32),
        ),
        grid=(_G,),
        in_specs=[
            pl.BlockSpec((6, _RB, _LANE), lambda g: (0, g, 0)),
            pl.BlockSpec((6, _RB, _LANE), lambda g: (0, g, 0)),
        ],
        out_specs=(
            pl.BlockSpec((_RB, _LANE), lambda g: (g, 0)),
            pl.BlockSpec((1, 1, _LANE), lambda g: (g, 0, 0)),
        ),
        compiler_params=pltpu.CompilerParams(
            dimension_semantics=("arbitrary",),
        ),
    )(p, t)
    iou = iou2d.reshape(_N)
    giou_loss = jnp.sum(parts).reshape(1)
    return iou, giou_loss


# speedup vs baseline: 118.2752x; 1.0246x over previous
import jax
import jax.numpy as jnp
from jax.experimental import pallas as pl
from jax.experimental.pallas import tpu as pltpu

_N = 131072
_LANE = 128
_ROWS = _N // _LANE
_RB = 32
_G = _ROWS // _RB
_K = 8


def _corner_lists(x, y, w, l, im, re):
    inv = jax.lax.rsqrt(im * im + re * re)
    c = re * inv
    s = im * inv
    hw = w * 0.5
    hl = l * 0.5
    hwc = hw * c
    hws = hw * s
    hlc = hl * c
    hls = hl * s
    cx = [x - hwc - hls, x - hwc + hls, x + hwc + hls, x + hwc - hls]
    cy = [y - hws + hlc, y - hws - hlc, y + hws - hlc, y + hws + hlc]
    return cx, cy


def _clip_inter_area(sx, sy, cx, cy):
    z = jnp.zeros_like(sx[0])
    one = jnp.ones_like(z)
    px = list(sx) + [z, z, z, z]
    py = list(sy) + [z, z, z, z]
    mval = [one, one, one, one, z, z, z, z]
    m = one * 4.0
    frozen = z
    for e in range(4):
        pX, pY = cx[e], cy[e]
        qX, qY = cx[(e + 1) % 4], cy[(e + 1) % 4]
        a = qY - pY
        b = pX - qX
        c = qX * pY - qY * pX
        cap_prev = 4 + e
        cap_new = min(5 + e, _K)
        vals = [a * px[k] + b * py[k] + c for k in range(cap_prev)]
        candx, candy, kv = [], [], []
        for k in range(cap_prev):
            if e == 0:
                k2 = (k + 1) % 4
                tpx, tpy, tv = px[k2], py[k2], vals[k2]
                mk = one
            else:
                if k + 1 < cap_prev:
                    wrap = m <= (k + 1)
                    tpx = jnp.where(wrap, px[0], px[k + 1])
                    tpy = jnp.where(wrap, py[0], py[k + 1])
                    tv = jnp.where(wrap, vals[0], vals[k + 1])
                else:
                    tpx, tpy, tv = px[0], py[0], vals[0]
                mk = mval[k]
            cross_cmp = vals[k] * tv < 0
            keep01 = jnp.where(vals[k] <= 0, mk, z)
            cross01 = jnp.where(cross_cmp, mk, z)
            den = jnp.where(cross_cmp, vals[k] - tv, 1.0)
            tpar = vals[k] / den
            ix = px[k] + tpar * (tpx - px[k])
            iy = py[k] + tpar * (tpy - py[k])
            candx += [px[k], ix]
            candy += [py[k], iy]
            kv += [keep01, cross01]
        ncand = len(kv)
        run = z
        for v in kv:
            run = run + v
        if e == 0:
            do01 = one
        else:
            do01 = jnp.where(frozen > 0.5, z, jnp.where(m > 2.5, one, z))
        accept01 = jnp.where(run > 0.5, do01, z)
        acc_b = accept01 > 0.5
        if e < 3:
            run_incl = z
            keys = []
            for v in kv:
                run_incl = run_incl + v
                keys.append(run_incl * v)
            new_px, new_py = [], []
            for j in range(cap_new):
                nx, ny = z, z
                for ci in range(j, ncand):
                    hit = keys[ci] == (j + 1.0)
                    nx = jnp.where(hit, candx[ci], nx)
                    ny = jnp.where(hit, candy[ci], ny)
                new_px.append(nx)
                new_py.append(ny)
            for j in range(cap_new):
                px[j] = jnp.where(acc_b, new_px[j], px[j])
                py[j] = jnp.where(acc_b, new_py[j], py[j])
                nmv = jnp.where(run > (j + 0.5), one, z)
                mval[j] = jnp.where(acc_b, nmv, mval[j])
            m = jnp.where(acc_b, jnp.minimum(run, float(cap_new)), m)
            frozen = jnp.maximum(frozen, jnp.where(run < 0.5, do01, z))
        else:
            kvb = [v > 0.5 for v in kv]
            fcx, fcy = z, z
            for ci in reversed(range(ncand)):
                fcx = jnp.where(kvb[ci], candx[ci], fcx)
                fcy = jnp.where(kvb[ci], candy[ci], fcy)
            carry_x, carry_y = fcx, fcy
            sacc = z
            for ci in reversed(range(ncand)):
                contrib = candx[ci] * carry_y - candy[ci] * carry_x
                sacc = sacc + jnp.where(kvb[ci], contrib, z)
                carry_x = jnp.where(kvb[ci], candx[ci], carry_x)
                carry_y = jnp.where(kvb[ci], candy[ci], carry_y)
            area_new = 0.5 * jnp.abs(sacc)
            fx, fy = px[0], py[0]
            spx = [jnp.where(mval[k] > 0.5, px[k], fx) for k in range(cap_prev)]
            spy = [jnp.where(mval[k] > 0.5, py[k], fy) for k in range(cap_prev)]
            pacc = z
            for k in range(cap_prev):
                k2 = (k + 1) % cap_prev
                pacc = pacc + (spx[k] * spy[k2] - spy[k] * spx[k2])
            area_prev = 0.5 * jnp.abs(pacc)
            m_fin = jnp.where(acc_b, jnp.minimum(run, float(cap_new)), m)
            area = jnp.where(acc_b, area_new, area_prev)
            return jnp.where(m_fin > 2.5, area, z)


def _hull_area8(hx, hy):
    z = jnp.zeros_like(hx[0])
    cr = {}
    for a in range(_K):
        for b in range(a + 1, _K):
            cr[(a, b)] = hx[a] * hy[b] - hy[a] * hx[b]

    def _crs(a, b):
        return cr[(a, b)] if a < b else -cr[(b, a)]

    total = z
    for i in range(_K):
        for j in range(i + 1, _K):
            same = (i < 4) == (j < 4)
            if same and (j - i) % 4 == 2:
                continue
            crij = cr[(i, j)]
            if same:
                quad = range(0, 4) if i < 4 else range(4, 8)
                cycle_fwd = (j - i == 1)
                ks = [k for k in range(_K) if k not in quad]
            else:
                cycle_fwd = None
                ks = [k for k in range(_K) if k != i and k != j]
            mn = None
            mx = None
            for k in ks:
                c = crij + _crs(j, k) + _crs(k, i)
                if cycle_fwd is not False:
                    mn = c if mn is None else jnp.minimum(mn, c)
                if cycle_fwd is not True:
                    mx = c if mx is None else jnp.maximum(mx, c)
            if same:
                if cycle_fwd:
                    total = total + jnp.where(mn >= -1e-6, crij, 0.0)
                else:
                    total = total - jnp.where(-mx >= -1e-6, crij, 0.0)
            else:
                fwd = jnp.where(mn >= -1e-6, crij, 0.0)
                bwd = jnp.where(-mx >= -1e-6, crij, 0.0)
                total = total + (fwd - bwd)
    return 0.5 * jnp.abs(total)


def _giou_block(p_ref, t_ref, iou_ref, loss_ref, acc_ref):
    xp, yp, wp, lp, imp, rep = (p_ref[i] for i in range(6))
    xt, yt, wt, lt, imt, ret = (t_ref[i] for i in range(6))
    pcx, pcy = _corner_lists(xp, yp, wp, lp, imp, rep)
    tcx, tcy = _corner_lists(xt, yt, wt, lt, imt, ret)
    inter = _clip_inter_area(pcx, pcy, tcx, tcy)
    p_area = wp * lp
    t_area = wt * lt
    union = p_area + t_area - inter
    iou = inter / (union + 1e-16)
    hull = _hull_area8(pcx + tcx, pcy + tcy)
    giou = 1.0 - (iou - (hull - union) / (hull + 1e-16))
    iou_ref[...] = iou
    s = jnp.sum(giou, axis=0, keepdims=True)
    g = pl.program_id(0)
    sacc = jnp.where(g == 0, s, acc_ref[...] + s)
    acc_ref[...] = sacc

    @pl.when(g == _G - 1)
    def _():
        loss_ref[...] = jnp.sum(sacc, keepdims=True)


def kernel(pred_boxes, target_boxes):
    p = pred_boxes.T.reshape(6, _ROWS, _LANE)
    t = target_boxes.T.reshape(6, _ROWS, _LANE)
    iou2d, loss = pl.pallas_call(
        _giou_block,
        out_shape=(
            jax.ShapeDtypeStruct((_ROWS, _LANE), jnp.float32),
            jax.ShapeDtypeStruct((1, 1), jnp.float32),
        ),
        grid=(_G,),
        in_specs=[
            pl.BlockSpec((6, _RB, _LANE), lambda g: (0, g, 0)),
            pl.BlockSpec((6, _RB, _LANE), lambda g: (0, g, 0)),
        ],
        out_specs=(
            pl.BlockSpec((_RB, _LANE), lambda g: (g, 0)),
            pl.BlockSpec((1, 1), lambda g: (0, 0)),
        ),
        scratch_shapes=[pltpu.VMEM((1, _LANE), jnp.float32)],
        compiler_params=pltpu.CompilerParams(
            dimension_semantics=("arbitrary",),
        ),
    )(p, t)
    iou = iou2d.reshape(_N)
    giou_loss = loss.reshape(1)
    return iou, giou_loss
